# Initial kernel scaffold; baseline (speedup 1.0000x reference)
import jax
import jax.numpy as jnp
from jax import lax
from jax.experimental import pallas as pl
from jax.experimental.pallas import tpu as pltpu
from jax.experimental.pallas import tpu_sc as plsc

B = 128
BLEN_PAD = 864


def _tc_taps(feats, wcat, c_out, center):
    n, c_in = feats.shape
    kkc = wcat.shape[1]
    bn = 2048
    grid = (pl.cdiv(n, bn),)

    def body(f_ref, w_ref, y_ref, yc_ref):
        y = jnp.dot(f_ref[...], w_ref[...], preferred_element_type=jnp.float32)
        y_ref[...] = y
        yc_ref[...] = y[:, center * c_out:(center + 1) * c_out]

    return pl.pallas_call(
        body,
        grid=grid,
        in_specs=[
            pl.BlockSpec((bn, c_in), lambda i: (i, 0)),
            pl.BlockSpec((c_in, kkc), lambda i: (0, 0)),
        ],
        out_specs=[
            pl.BlockSpec((bn, kkc), lambda i: (i, 0)),
            pl.BlockSpec((bn, c_out), lambda i: (i, 0)),
        ],
        out_shape=[
            jax.ShapeDtypeStruct((n, kkc), jnp.float32),
            jax.ShapeDtypeStruct((n, c_out), jnp.float32),
        ],
    )(feats, wcat)


def _sc_scatter(y2, yc, imap_p, omap_p, b_pad, n, c_out, nw, rw, nseg):
    mesh = plsc.VectorSubcoreMesh(core_axis_name="c", subcore_axis_name="s",
                                  num_cores=2, num_subcores=16)

    def body(y2_h, yc_h, imap_h, omap_h, b_h, out_h,
             b_v, idx_i, idx_o, gbuf, acc, sem):
        c = lax.axis_index("c")
        s = lax.axis_index("s")
        w = s * 2 + c
        base_row = w * rw
        pltpu.sync_copy(b_h, b_v)
        pltpu.sync_copy(yc_h.at[pl.ds(base_row, rw)], acc.at[pl.ds(0, rw)])
        lanes = lax.iota(jnp.int32, 16)

        def seg_body(j, carry):
            start = b_v[j * (nw + 1) + w]
            end = b_v[j * (nw + 1) + w + 1]
            a = start - lax.rem(start, 8)
            nblocks = lax.div(end - a + (B - 1), B)

            def blk(t, carry2):
                p = pl.multiple_of(a + t * B, 8)
                pltpu.sync_copy(imap_h.at[pl.ds(p, B)], idx_i)
                pltpu.sync_copy(omap_h.at[pl.ds(p, B)], idx_o)
                for u in range(B // 16):
                    posv = p + u * 16 + lanes
                    m = (posv >= start) & (posv < end)
                    ov = idx_o[pl.ds(u * 16, 16)]
                    idx_o[pl.ds(u * 16, 16)] = jnp.where(m, ov, rw)
                pltpu.async_copy(y2_h.at[idx_i], gbuf, sem).wait()
                pltpu.sync_copy(gbuf, acc.at[idx_o], add=True)
                return carry2

            return lax.fori_loop(0, nblocks, blk, carry)

        lax.fori_loop(0, nseg, seg_body, 0)
        pltpu.sync_copy(acc.at[pl.ds(0, rw)], out_h.at[pl.ds(base_row, rw)])

    return pl.kernel(
        body,
        out_type=jax.ShapeDtypeStruct((n, c_out), jnp.float32),
        mesh=mesh,
        scratch_types=[
            pltpu.VMEM((BLEN_PAD,), jnp.int32),
            pltpu.VMEM((B,), jnp.int32),
            pltpu.VMEM((B,), jnp.int32),
            pltpu.VMEM((B, c_out), jnp.float32),
            pltpu.VMEM((rw + 1, c_out), jnp.float32),
            pltpu.SemaphoreType.DMA,
        ],
    )(y2, yc, imap_p, omap_p, b_pad)


def kernel(feats, kernel, imap, omap, kpos):
    n, c_in = feats.shape
    kk, _, c_out = kernel.shape
    center = (kk - 1) // 2
    nseg = kk - 1
    m = imap.shape[0]
    nw = 32
    while n % nw:
        nw //= 2
    rw = n // nw

    wcat = kernel.transpose(1, 0, 2).reshape(c_in, kk * c_out)
    y, yc = _tc_taps(feats, wcat, c_out, center)
    y2 = y.reshape(n * kk, c_out)

    pos = jnp.arange(m, dtype=jnp.int32)
    seg = jnp.searchsorted(kpos, pos, side="right").astype(jnp.int32) - 1
    k_of = jnp.where(seg < center, seg, seg + 1)
    imap_adj = imap * kk + k_of
    omap_mod = jnp.remainder(omap, rw)
    key = seg * n + omap
    targets = (jnp.arange(nseg, dtype=jnp.int32)[:, None] * n
               + jnp.minimum(jnp.arange(nw + 1, dtype=jnp.int32)[None, :] * rw, n)
               ).reshape(-1)
    b = jnp.searchsorted(key, targets).astype(jnp.int32)
    b_pad = jnp.pad(b, (0, BLEN_PAD - nseg * (nw + 1)))
    imap_p = jnp.pad(imap_adj, (0, B))
    omap_p = jnp.pad(omap_mod, (0, B))

    return _sc_scatter(y2, yc, imap_p, omap_p, b_pad, n, c_out, nw, rw, nseg)


# speedup vs baseline: 59.9690x; 59.9690x over previous
import jax
import jax.numpy as jnp
from jax import lax
from jax.experimental import pallas as pl
from jax.experimental.pallas import tpu as pltpu
from jax.experimental.pallas import tpu_sc as plsc

B = 128
BLEN_PAD = 896


def _tc_taps(feats, wcat, c_out, center, n_pad):
    n, c_in = feats.shape
    kkc = wcat.shape[1]
    bn = 2048
    grid = (pl.cdiv(n_pad, bn),)

    def body(f_ref, w_ref, y_ref, yc_ref):
        y = jnp.dot(f_ref[...], w_ref[...], preferred_element_type=jnp.float32)
        y_ref[...] = y
        yc_ref[...] = y[:, center * c_out:(center + 1) * c_out]

    return pl.pallas_call(
        body,
        grid=grid,
        in_specs=[
            pl.BlockSpec((bn, c_in), lambda i: (i, 0)),
            pl.BlockSpec((c_in, kkc), lambda i: (0, 0)),
        ],
        out_specs=[
            pl.BlockSpec((bn, kkc), lambda i: (i, 0)),
            pl.BlockSpec((bn, c_out), lambda i: (i, 0)),
        ],
        out_shape=[
            jax.ShapeDtypeStruct((n, kkc), jnp.float32),
            jax.ShapeDtypeStruct((n_pad, c_out), jnp.float32),
        ],
    )(feats, wcat)


def _sc_scatter(y2, yc, imap_p, omap_p, b_pad, n_pad, c_out, nw, rw, nseg):
    mesh = plsc.VectorSubcoreMesh(core_axis_name="c", subcore_axis_name="s",
                                  num_cores=2, num_subcores=16)
    h = n_pad // 2

    def body(y2_h, yc_h, imap_h, omap_h, b_h, out_h,
             b_v, idx_i, idx_o, gbuf, acc, sem):
        c = lax.axis_index("c")
        s = lax.axis_index("s")
        w = c * 16 + s
        base_row = w * rw
        pltpu.sync_copy(b_h, b_v)
        pltpu.sync_copy(yc_h.at[pl.ds(base_row, rw)],
                        acc.at[pl.ds(s * rw, rw)])
        lanes = lax.iota(jnp.int32, 16)

        def _scalar_at(i):
            v = b_v[pl.ds(i, 16)]
            return v[0]

        def seg_body(j, carry):
            start = _scalar_at(j * (nw + 1) + w)
            end = _scalar_at(j * (nw + 1) + w + 1)
            a = start - lax.rem(start, 8)
            nblocks = lax.div(end - a + (B - 1), B)

            def blk(t, carry2):
                p = pl.multiple_of(a + t * B, 8)
                pltpu.sync_copy(imap_h.at[pl.ds(p, B)], idx_i)
                pltpu.sync_copy(omap_h.at[pl.ds(p, B)], idx_o)
                for u in range(B // 16):
                    posv = p + u * 16 + lanes
                    m = (posv >= start) & (posv < end)
                    ov = idx_o[pl.ds(u * 16, 16)]
                    idx_o[pl.ds(u * 16, 16)] = jnp.where(m, ov, h)
                pltpu.async_copy(y2_h.at[idx_i], gbuf, sem).wait()
                pltpu.sync_copy(gbuf, acc.at[idx_o], add=True)
                return carry2

            return lax.fori_loop(0, nblocks, blk, carry)

        lax.fori_loop(0, nseg, seg_body, 0)
        pltpu.sync_copy(acc.at[pl.ds(s * rw, rw)],
                        out_h.at[pl.ds(base_row, rw)])

    return pl.kernel(
        body,
        out_type=jax.ShapeDtypeStruct((n_pad, c_out), jnp.float32),
        mesh=mesh,
        compiler_params=pltpu.CompilerParams(use_tc_tiling_on_sc=False),
        scratch_types=[
            pltpu.VMEM((BLEN_PAD,), jnp.int32),
            pltpu.VMEM((B,), jnp.int32),
            pltpu.VMEM((B,), jnp.int32),
            pltpu.VMEM((B, c_out), jnp.float32),
            pltpu.VMEM_SHARED((h + 8, c_out), jnp.float32),
            pltpu.SemaphoreType.DMA,
        ],
    )(y2, yc, imap_p, omap_p, b_pad)


def kernel(feats, kernel, imap, omap, kpos):
    n, c_in = feats.shape
    kk, _, c_out = kernel.shape
    center = (kk - 1) // 2
    nseg = kk - 1
    m = imap.shape[0]
    nw = 32
    n_pad = -(-n // 256) * 256
    rw = n_pad // nw

    wcat = kernel.transpose(1, 0, 2).reshape(c_in, kk * c_out)
    y, yc = _tc_taps(feats, wcat, c_out, center, n_pad)
    y2 = y.reshape(n * kk, c_out)

    pos = jnp.arange(m, dtype=jnp.int32)
    seg = jnp.searchsorted(kpos, pos, side="right").astype(jnp.int32) - 1
    k_of = jnp.where(seg < center, seg, seg + 1)
    imap_adj = imap * kk + k_of
    omap_mod = jnp.remainder(omap, n_pad // 2)
    key = seg * n + omap
    targets = (jnp.arange(nseg, dtype=jnp.int32)[:, None] * n
               + jnp.minimum(jnp.arange(nw + 1, dtype=jnp.int32)[None, :] * rw, n)
               ).reshape(-1)
    b = jnp.searchsorted(key, targets).astype(jnp.int32)
    b_pad = jnp.pad(b, (0, BLEN_PAD - nseg * (nw + 1)))
    imap_p = jnp.pad(imap_adj, (0, B))
    omap_p = jnp.pad(omap_mod, (0, B))

    out = _sc_scatter(y2, yc, imap_p, omap_p, b_pad, n_pad, c_out, nw, rw, nseg)
    return out[:n]
